# baseline (speedup 1.0000x reference)
import jax
import jax.numpy as jnp
from jax.experimental import pallas as pl
from jax.experimental.pallas import tpu as pltpu

N = 10000
D_IN = 256
D_OUT = 64

_HI = jax.lax.Precision.HIGHEST


def _pre_body(feat_ref, feata_ref, w1_ref, zcat_ref):
    w1 = w1_ref[...].astype(jnp.bfloat16)
    zcat_ref[:, :D_OUT] = jnp.dot(feat_ref[...].astype(jnp.bfloat16), w1,
                                  preferred_element_type=jnp.float32)
    zcat_ref[:, D_OUT:] = jnp.dot(feata_ref[...].astype(jnp.bfloat16), w1,
                                  preferred_element_type=jnp.float32)


def _pass1_body(adj_ref, adja_ref, zcat_ref, w2_ref, z_ref, za_ref, zs_ref,
                mean_ref, disp_ref):
    i = pl.program_id(0)
    nrows = z_ref.shape[0]
    zcat = zcat_ref[...]
    y1 = jnp.dot(adj_ref[...], zcat, preferred_element_type=jnp.float32)
    z_ref[...] = y1[:, :D_OUT]
    za_ref[...] = y1[:, D_OUT:]
    zs_ref[...] = jnp.dot(adja_ref[...], zcat[:, D_OUT:],
                          preferred_element_type=jnp.float32)
    z0_rows = zcat_ref[pl.ds(i * nrows, nrows), :D_OUT].astype(jnp.bfloat16)
    zinb = jnp.dot(z0_rows, w2_ref[...].astype(jnp.bfloat16),
                   preferred_element_type=jnp.float32)
    mean_ref[...] = jnp.clip(jnp.exp(zinb), 1e-5, 1e6)
    disp_ref[...] = jnp.clip(jax.nn.softplus(zinb), 1e-4, 1e4)


def _pass2_body(adjt_ref, adjb_ref, zfull_ref, za_ref, zs_ref, w2_ref,
                dw_ref, db_ref, h_ref, ret_ref, reta_ref):
    i = pl.program_id(0)
    nrows = h_ref.shape[0]
    half = nrows // 2
    zfull = zfull_ref[...]
    w2 = w2_ref[...]
    dw = dw_ref[...]
    b = db_ref[0, 0]

    for k, adj_ref in ((0, adjt_ref), (1, adjb_ref)):
        off = k * half
        z2 = jnp.dot(adj_ref[...], zfull, preferred_element_type=jnp.float32)
        h_ref[off:off + half, :] = jnp.dot(
            z2, w2, precision=_HI, preferred_element_type=jnp.float32)
        emb = jax.nn.relu(zfull_ref[pl.ds(i * nrows + off, half), :])
        emb_a = jax.nn.relu(za_ref[off:off + half, :])
        emb_s = jax.nn.relu(zs_ref[off:off + half, :])
        t = jnp.dot(emb_a, dw, precision=_HI,
                    preferred_element_type=jnp.float32)
        ret_ref[off:off + half, :] = jnp.sum(t * emb, axis=1,
                                             keepdims=True) + b
        t2 = jnp.dot(emb_s, dw, precision=_HI,
                     preferred_element_type=jnp.float32)
        reta_ref[off:off + half, :] = jnp.sum(t2 * emb_a, axis=1,
                                              keepdims=True) + b


def kernel(feat, feat_a, adj, adj_a, W1, W2, disc_W, disc_b):
    BN = 1000
    BI1 = 200
    BI2 = 400

    f32 = jnp.float32

    zcat = pl.pallas_call(
        _pre_body,
        grid=(N // BN,),
        in_specs=[
            pl.BlockSpec((BN, D_IN), lambda i: (i, 0)),
            pl.BlockSpec((BN, D_IN), lambda i: (i, 0)),
            pl.BlockSpec((D_IN, D_OUT), lambda i: (0, 0)),
        ],
        out_specs=pl.BlockSpec((BN, 2 * D_OUT), lambda i: (i, 0)),
        out_shape=jax.ShapeDtypeStruct((N, 2 * D_OUT), f32),
        compiler_params=pltpu.CompilerParams(
            dimension_semantics=("parallel",)),
    )(feat, feat_a, W1)

    z, za, zs, mean, disp = pl.pallas_call(
        _pass1_body,
        grid=(N // BI1,),
        in_specs=[
            pl.BlockSpec((BI1, N), lambda i: (i, 0)),
            pl.BlockSpec((BI1, N), lambda i: (i, 0)),
            pl.BlockSpec((N, 2 * D_OUT), lambda i: (0, 0)),
            pl.BlockSpec((D_OUT, D_IN), lambda i: (0, 0)),
        ],
        out_specs=[
            pl.BlockSpec((BI1, D_OUT), lambda i: (i, 0)),
            pl.BlockSpec((BI1, D_OUT), lambda i: (i, 0)),
            pl.BlockSpec((BI1, D_OUT), lambda i: (i, 0)),
            pl.BlockSpec((BI1, D_IN), lambda i: (i, 0)),
            pl.BlockSpec((BI1, D_IN), lambda i: (i, 0)),
        ],
        out_shape=[
            jax.ShapeDtypeStruct((N, D_OUT), f32),
            jax.ShapeDtypeStruct((N, D_OUT), f32),
            jax.ShapeDtypeStruct((N, D_OUT), f32),
            jax.ShapeDtypeStruct((N, D_IN), f32),
            jax.ShapeDtypeStruct((N, D_IN), f32),
        ],
        compiler_params=pltpu.CompilerParams(
            dimension_semantics=("parallel",)),
    )(adj, adj_a, zcat, W2)

    h, ret, ret_a = pl.pallas_call(
        _pass2_body,
        grid=(N // BI2,),
        in_specs=[
            pl.BlockSpec((BI2 // 2, N), lambda i: (2 * i, 0)),
            pl.BlockSpec((BI2 // 2, N), lambda i: (2 * i + 1, 0)),
            pl.BlockSpec((N, D_OUT), lambda i: (0, 0)),
            pl.BlockSpec((BI2, D_OUT), lambda i: (i, 0)),
            pl.BlockSpec((BI2, D_OUT), lambda i: (i, 0)),
            pl.BlockSpec((D_OUT, D_IN), lambda i: (0, 0)),
            pl.BlockSpec((D_OUT, D_OUT), lambda i: (0, 0)),
            pl.BlockSpec((1, 1), lambda i: (0, 0)),
        ],
        out_specs=[
            pl.BlockSpec((BI2, D_IN), lambda i: (i, 0)),
            pl.BlockSpec((BI2, 1), lambda i: (i, 0)),
            pl.BlockSpec((BI2, 1), lambda i: (i, 0)),
        ],
        out_shape=[
            jax.ShapeDtypeStruct((N, D_IN), f32),
            jax.ShapeDtypeStruct((N, 1), f32),
            jax.ShapeDtypeStruct((N, 1), f32),
        ],
        compiler_params=pltpu.CompilerParams(
            dimension_semantics=("parallel",)),
    )(adj, adj, z, za, zs, W2, disc_W.reshape(D_OUT, D_OUT),
      disc_b.reshape(1, 1))

    return (z, h, ret, ret_a, mean, disp)


# speedup vs baseline: 1.0652x; 1.0652x over previous
import jax
import jax.numpy as jnp
from jax.experimental import pallas as pl
from jax.experimental.pallas import tpu as pltpu

N = 10000
D_IN = 256
D_OUT = 64

_HI = jax.lax.Precision.HIGHEST


def _pre_body(feat_ref, feata_ref, w1_ref, zcat_ref):
    w1 = w1_ref[...].astype(jnp.bfloat16)
    zcat_ref[:, :D_OUT] = jnp.dot(feat_ref[...].astype(jnp.bfloat16), w1,
                                  preferred_element_type=jnp.float32)
    zcat_ref[:, D_OUT:] = jnp.dot(feata_ref[...].astype(jnp.bfloat16), w1,
                                  preferred_element_type=jnp.float32)


def _pass1_body(adj_ref, zcat_ref, w2_ref, z_ref, za_ref, mean_ref,
                disp_ref):
    i = pl.program_id(0)
    nrows = z_ref.shape[0]
    zcat = zcat_ref[...]
    y1 = jnp.dot(adj_ref[...], zcat, preferred_element_type=jnp.float32)
    z_ref[...] = y1[:, :D_OUT]
    za_ref[...] = y1[:, D_OUT:]
    z0_rows = zcat_ref[pl.ds(i * nrows, nrows), :D_OUT].astype(jnp.bfloat16)
    zinb = jnp.dot(z0_rows, w2_ref[...].astype(jnp.bfloat16),
                   preferred_element_type=jnp.float32)
    mean_ref[...] = jnp.clip(jnp.exp(zinb), 1e-5, 1e6)
    disp_ref[...] = jnp.clip(jax.nn.softplus(zinb), 1e-4, 1e4)


def _passa_body(adja_ref, zcat_ref, zs_ref):
    zs_ref[...] = jnp.dot(adja_ref[...], zcat_ref[:, D_OUT:],
                          preferred_element_type=jnp.float32)


def _pass2_body(adj_ref, zfull_ref, za_ref, zs_ref, w2_ref, dw_ref,
                db_ref, h_ref, ret_ref, reta_ref):
    i = pl.program_id(0)
    nrows = h_ref.shape[0]
    z2 = jnp.dot(adj_ref[...], zfull_ref[...],
                 preferred_element_type=jnp.float32)
    h_ref[...] = jnp.dot(z2, w2_ref[...], precision=_HI,
                         preferred_element_type=jnp.float32)
    emb = jax.nn.relu(zfull_ref[pl.ds(i * nrows, nrows), :])
    emb_a = jax.nn.relu(za_ref[...])
    emb_s = jax.nn.relu(zs_ref[...])
    dw = dw_ref[...]
    b = db_ref[0, 0]
    t = jnp.dot(emb_a, dw, precision=_HI, preferred_element_type=jnp.float32)
    ret_ref[...] = jnp.sum(t * emb, axis=1, keepdims=True) + b
    t2 = jnp.dot(emb_s, dw, precision=_HI, preferred_element_type=jnp.float32)
    reta_ref[...] = jnp.sum(t2 * emb_a, axis=1, keepdims=True) + b


def kernel(feat, feat_a, adj, adj_a, W1, W2, disc_W, disc_b):
    BN = 1000
    BI1 = 400
    BI2 = 400

    f32 = jnp.float32

    zcat = pl.pallas_call(
        _pre_body,
        grid=(N // BN,),
        in_specs=[
            pl.BlockSpec((BN, D_IN), lambda i: (i, 0)),
            pl.BlockSpec((BN, D_IN), lambda i: (i, 0)),
            pl.BlockSpec((D_IN, D_OUT), lambda i: (0, 0)),
        ],
        out_specs=pl.BlockSpec((BN, 2 * D_OUT), lambda i: (i, 0)),
        out_shape=jax.ShapeDtypeStruct((N, 2 * D_OUT), f32),
        compiler_params=pltpu.CompilerParams(
            dimension_semantics=("parallel",)),
    )(feat, feat_a, W1)

    z, za, mean, disp = pl.pallas_call(
        _pass1_body,
        grid=(N // BI1,),
        in_specs=[
            pl.BlockSpec((BI1, N), lambda i: (i, 0)),
            pl.BlockSpec((N, 2 * D_OUT), lambda i: (0, 0)),
            pl.BlockSpec((D_OUT, D_IN), lambda i: (0, 0)),
        ],
        out_specs=[
            pl.BlockSpec((BI1, D_OUT), lambda i: (i, 0)),
            pl.BlockSpec((BI1, D_OUT), lambda i: (i, 0)),
            pl.BlockSpec((BI1, D_IN), lambda i: (i, 0)),
            pl.BlockSpec((BI1, D_IN), lambda i: (i, 0)),
        ],
        out_shape=[
            jax.ShapeDtypeStruct((N, D_OUT), f32),
            jax.ShapeDtypeStruct((N, D_OUT), f32),
            jax.ShapeDtypeStruct((N, D_IN), f32),
            jax.ShapeDtypeStruct((N, D_IN), f32),
        ],
        compiler_params=pltpu.CompilerParams(
            dimension_semantics=("parallel",)),
    )(adj, zcat, W2)

    zs = pl.pallas_call(
        _passa_body,
        grid=(N // BI1,),
        in_specs=[
            pl.BlockSpec((BI1, N), lambda i: (i, 0)),
            pl.BlockSpec((N, 2 * D_OUT), lambda i: (0, 0)),
        ],
        out_specs=pl.BlockSpec((BI1, D_OUT), lambda i: (i, 0)),
        out_shape=jax.ShapeDtypeStruct((N, D_OUT), f32),
        compiler_params=pltpu.CompilerParams(
            dimension_semantics=("parallel",)),
    )(adj_a, zcat)

    h, ret, ret_a = pl.pallas_call(
        _pass2_body,
        grid=(N // BI2,),
        in_specs=[
            pl.BlockSpec((BI2, N), lambda i: (i, 0)),
            pl.BlockSpec((N, D_OUT), lambda i: (0, 0)),
            pl.BlockSpec((BI2, D_OUT), lambda i: (i, 0)),
            pl.BlockSpec((BI2, D_OUT), lambda i: (i, 0)),
            pl.BlockSpec((D_OUT, D_IN), lambda i: (0, 0)),
            pl.BlockSpec((D_OUT, D_OUT), lambda i: (0, 0)),
            pl.BlockSpec((1, 1), lambda i: (0, 0)),
        ],
        out_specs=[
            pl.BlockSpec((BI2, D_IN), lambda i: (i, 0)),
            pl.BlockSpec((BI2, 1), lambda i: (i, 0)),
            pl.BlockSpec((BI2, 1), lambda i: (i, 0)),
        ],
        out_shape=[
            jax.ShapeDtypeStruct((N, D_IN), f32),
            jax.ShapeDtypeStruct((N, 1), f32),
            jax.ShapeDtypeStruct((N, 1), f32),
        ],
        compiler_params=pltpu.CompilerParams(
            dimension_semantics=("parallel",)),
    )(adj, z, za, zs, W2, disc_W.reshape(D_OUT, D_OUT),
      disc_b.reshape(1, 1))

    return (z, h, ret, ret_a, mean, disp)


# speedup vs baseline: 1.0921x; 1.0253x over previous
import jax
import jax.numpy as jnp
from jax.experimental import pallas as pl
from jax.experimental.pallas import tpu as pltpu

N = 10000
D_IN = 256
D_OUT = 64

_HI = jax.lax.Precision.HIGHEST


def _pre_body(feat_ref, feata_ref, w1_ref, zcat_ref):
    w1 = w1_ref[...].astype(jnp.bfloat16)
    zcat_ref[:, :D_OUT] = jnp.dot(feat_ref[...].astype(jnp.bfloat16), w1,
                                  preferred_element_type=jnp.float32)
    zcat_ref[:, D_OUT:] = jnp.dot(feata_ref[...].astype(jnp.bfloat16), w1,
                                  preferred_element_type=jnp.float32)


def _pass1_body(adj_ref, adja_ref, zcat_ref, w2_ref, z_ref, za_ref, zs_ref,
                mean_ref, disp_ref):
    i = pl.program_id(0)
    nrows = z_ref.shape[0]
    zcat = zcat_ref[...]
    y1 = jnp.dot(adj_ref[...], zcat, preferred_element_type=jnp.float32)
    z_ref[...] = y1[:, :D_OUT]
    za_ref[...] = y1[:, D_OUT:]
    zs_ref[...] = jnp.dot(adja_ref[...], zcat[:, D_OUT:],
                          preferred_element_type=jnp.float32)
    z0_rows = zcat_ref[pl.ds(i * nrows, nrows), :D_OUT].astype(jnp.bfloat16)
    zinb = jnp.dot(z0_rows, w2_ref[...].astype(jnp.bfloat16),
                   preferred_element_type=jnp.float32)
    mean_ref[...] = jnp.clip(jnp.exp(zinb), 1e-5, 1e6)
    disp_ref[...] = jnp.clip(jax.nn.softplus(zinb), 1e-4, 1e4)


def _pass2_body(adjt_ref, adjb_ref, zfull_ref, za_ref, zs_ref, w2_ref,
                dw_ref, db_ref, h_ref, ret_ref, reta_ref):
    i = pl.program_id(0)
    nrows = h_ref.shape[0]
    zfull = zfull_ref[...]
    z2 = jnp.concatenate(
        [jnp.dot(adjt_ref[...], zfull, preferred_element_type=jnp.float32),
         jnp.dot(adjb_ref[...], zfull, preferred_element_type=jnp.float32)],
        axis=0)
    h_ref[...] = jnp.dot(z2, w2_ref[...], precision=_HI,
                         preferred_element_type=jnp.float32)
    emb = jax.nn.relu(zfull_ref[pl.ds(i * nrows, nrows), :])
    emb_a = jax.nn.relu(za_ref[...])
    emb_s = jax.nn.relu(zs_ref[...])
    dw = dw_ref[...]
    b = db_ref[0, 0]
    t = jnp.dot(emb_a, dw, precision=_HI, preferred_element_type=jnp.float32)
    ret_ref[...] = jnp.sum(t * emb, axis=1, keepdims=True) + b
    t2 = jnp.dot(emb_s, dw, precision=_HI, preferred_element_type=jnp.float32)
    reta_ref[...] = jnp.sum(t2 * emb_a, axis=1, keepdims=True) + b


def kernel(feat, feat_a, adj, adj_a, W1, W2, disc_W, disc_b):
    BN = 1000
    BI1 = 200
    BI2 = 400

    f32 = jnp.float32

    zcat = pl.pallas_call(
        _pre_body,
        grid=(N // BN,),
        in_specs=[
            pl.BlockSpec((BN, D_IN), lambda i: (i, 0)),
            pl.BlockSpec((BN, D_IN), lambda i: (i, 0)),
            pl.BlockSpec((D_IN, D_OUT), lambda i: (0, 0)),
        ],
        out_specs=pl.BlockSpec((BN, 2 * D_OUT), lambda i: (i, 0)),
        out_shape=jax.ShapeDtypeStruct((N, 2 * D_OUT), f32),
        compiler_params=pltpu.CompilerParams(
            dimension_semantics=("parallel",)),
    )(feat, feat_a, W1)

    z, za, zs, mean, disp = pl.pallas_call(
        _pass1_body,
        grid=(N // BI1,),
        in_specs=[
            pl.BlockSpec((BI1, N), lambda i: (i, 0)),
            pl.BlockSpec((BI1, N), lambda i: (i, 0)),
            pl.BlockSpec((N, 2 * D_OUT), lambda i: (0, 0)),
            pl.BlockSpec((D_OUT, D_IN), lambda i: (0, 0)),
        ],
        out_specs=[
            pl.BlockSpec((BI1, D_OUT), lambda i: (i, 0)),
            pl.BlockSpec((BI1, D_OUT), lambda i: (i, 0)),
            pl.BlockSpec((BI1, D_OUT), lambda i: (i, 0)),
            pl.BlockSpec((BI1, D_IN), lambda i: (i, 0)),
            pl.BlockSpec((BI1, D_IN), lambda i: (i, 0)),
        ],
        out_shape=[
            jax.ShapeDtypeStruct((N, D_OUT), f32),
            jax.ShapeDtypeStruct((N, D_OUT), f32),
            jax.ShapeDtypeStruct((N, D_OUT), f32),
            jax.ShapeDtypeStruct((N, D_IN), f32),
            jax.ShapeDtypeStruct((N, D_IN), f32),
        ],
        compiler_params=pltpu.CompilerParams(
            dimension_semantics=("parallel",)),
    )(adj, adj_a, zcat, W2)

    h, ret, ret_a = pl.pallas_call(
        _pass2_body,
        grid=(N // BI2,),
        in_specs=[
            pl.BlockSpec((BI2 // 2, N), lambda i: (2 * i, 0)),
            pl.BlockSpec((BI2 // 2, N), lambda i: (2 * i + 1, 0)),
            pl.BlockSpec((N, D_OUT), lambda i: (0, 0)),
            pl.BlockSpec((BI2, D_OUT), lambda i: (i, 0)),
            pl.BlockSpec((BI2, D_OUT), lambda i: (i, 0)),
            pl.BlockSpec((D_OUT, D_IN), lambda i: (0, 0)),
            pl.BlockSpec((D_OUT, D_OUT), lambda i: (0, 0)),
            pl.BlockSpec((1, 1), lambda i: (0, 0)),
        ],
        out_specs=[
            pl.BlockSpec((BI2, D_IN), lambda i: (i, 0)),
            pl.BlockSpec((BI2, 1), lambda i: (i, 0)),
            pl.BlockSpec((BI2, 1), lambda i: (i, 0)),
        ],
        out_shape=[
            jax.ShapeDtypeStruct((N, D_IN), f32),
            jax.ShapeDtypeStruct((N, 1), f32),
            jax.ShapeDtypeStruct((N, 1), f32),
        ],
        compiler_params=pltpu.CompilerParams(
            dimension_semantics=("parallel",)),
    )(adj, adj, z, za, zs, W2, disc_W.reshape(D_OUT, D_OUT),
      disc_b.reshape(1, 1))

    return (z, h, ret, ret_a, mean, disp)
